# baseline (speedup 1.0000x reference)
import jax
import jax.numpy as jnp
from jax import lax
from jax.experimental import pallas as pl
from jax.experimental.pallas import tpu as pltpu
from jax.experimental.pallas import tpu_sc as plsc

N_USER = 25000
N_ITEM = 25000
N = N_USER + N_ITEM
D = 64
NS = 16
L = 16
C = 96
SUP = 8
NB = 3
GLEAD = 2
NM = 3
N_PAD = 50176
NHALF = N_PAD // 2
R_PER_TILE = NHALF // NS
RC = 56
CAP_CH = 544
STG = 208
PK = plsc.PackFormat.INTERLEAVED


def _pack_row(fbuf, bbuf, r):
    for h in (0, 2 * L):
        p = plsc.pack(fbuf[r, pl.ds(h, L)], fbuf[r, pl.ds(h + L, L)],
                      format=PK)
        bbuf[r, pl.ds(h, 2 * L)] = p


def _p0_body(tab, src2, dst2, val2,
             tb, csrc, cdst, cval, counts,
             src_sv, dst_sv, val_sv, f_v, b_v,
             ss_v, sd_v, sv_v, i_s):
    cid = lax.axis_index("c")
    sid = lax.axis_index("s")
    wrow = cid * NS + sid
    crow0 = wrow * CAP_CH
    lo = cid * NHALF
    hi = lo + NHALF
    zi = jnp.zeros((L,), jnp.int32)
    zf = jnp.zeros((L,), jnp.float32)

    prow0 = wrow * (N_PAD // (2 * NS))

    def pt(i, carry):
        g0 = prow0 + i * RC
        pltpu.sync_copy(tab.at[pl.ds(g0, RC)], f_v.at[pl.ds(0, RC)])

        def pr(r, c2):
            _pack_row(f_v, b_v, r)
            return c2
        lax.fori_loop(0, RC, pr, 0)
        pltpu.sync_copy(b_v.at[pl.ds(0, RC)], tb.at[pl.ds(g0, RC)])
        return carry
    lax.fori_loop(0, (N_PAD // (2 * NS)) // RC, pt, 0)

    scan_e0 = sid * (src2.shape[0] // NS)
    n_scan_sup = (src2.shape[0] // NS) // (SUP * C)

    def flush(out_c):
        e0 = (crow0 + out_c) * C
        pltpu.sync_copy(ss_v.at[pl.ds(0, C)], csrc.at[pl.ds(e0, C)])
        pltpu.sync_copy(sd_v.at[pl.ds(0, C)], cdst.at[pl.ds(e0, C)])
        pltpu.sync_copy(sv_v.at[pl.ds(0, C)], cval.at[pl.ds(e0, C)])

    def scan_super(s, carry):
        e0 = scan_e0 + s * SUP * C
        pltpu.sync_copy(src2.at[pl.ds(e0, SUP * C)], src_sv)
        pltpu.sync_copy(dst2.at[pl.ds(e0, SUP * C)], dst_sv)
        pltpu.sync_copy(val2.at[pl.ds(e0, SUP * C)], val_sv)

        def scan_row(r, c2):
            cnt, out_c = c2
            for g in range(C // L):
                off = r * C + g * L
                d16 = dst_sv[pl.ds(off, L)]
                s16 = src_sv[pl.ds(off, L)]
                v16 = val_sv[pl.ds(off, L)]
                m = (d16 >= lo) & (d16 < hi)
                plsc.store_compressed(sd_v.at[pl.ds(cnt, L)], d16 - lo,
                                      mask=m)
                plsc.store_compressed(ss_v.at[pl.ds(cnt, L)], s16, mask=m)
                plsc.store_compressed(sv_v.at[pl.ds(cnt, L)], v16, mask=m)
                n = plsc.all_reduce_population_count(m)[0]
                cnt = cnt + n
                full = cnt >= C

                @pl.when(full)
                def _():
                    flush(out_c)
                    for st in (ss_v, sd_v, sv_v):
                        st[pl.ds(0, L)] = st[pl.ds(C, L)]
                cnt = jnp.where(full, cnt - C, cnt)
                out_c = out_c + full.astype(jnp.int32)
            return (cnt, out_c)
        return lax.fori_loop(0, SUP, scan_row, carry)

    cnt, out_c = lax.fori_loop(0, n_scan_sup, scan_super, (0, 0))

    for j in range(C // L):
        sd_v[pl.ds(cnt + j * L, L)] = zi
        ss_v[pl.ds(cnt + j * L, L)] = zi
        sv_v[pl.ds(cnt + j * L, L)] = zf
    flush(out_c)
    out_c = out_c + 1
    for j in range(C // L):
        sd_v[pl.ds(j * L, L)] = zi
        ss_v[pl.ds(j * L, L)] = zi
        sv_v[pl.ds(j * L, L)] = zf
    pad_n = (SUP - lax.rem(out_c, SUP)) % SUP

    def padb(i, c2):
        flush(out_c + i)
        return c2
    lax.fori_loop(0, pad_n, padb, 0)
    n_sup = (out_c + pad_n) // SUP
    sd_v[pl.ds(0, L)] = jnp.full((L,), n_sup, jnp.int32)
    pltpu.sync_copy(sd_v.at[pl.ds(0, L)], counts.at[pl.ds(wrow * L, L)])


def _make_layer(mean_mode):
  def _body(*args):
    if mean_mode:
        (tab, buf1, src_tab, csrc, cdst, cval, counts, out,
         acc, src_sv, dst_sv, val_sv,
         b0_v, b1_v, b2_v, m0_v, m1_v, m2_v,
         g0, g1, g2, s0, s1, s2, i_s) = args
    else:
        (src_tab, csrc, cdst, cval, counts, out,
         acc, src_sv, dst_sv, val_sv,
         b0_v, b1_v, b2_v, m0_v, m1_v, m2_v,
         g0, g1, g2, s0, s1, s2, i_s) = args
    cid = lax.axis_index("c")
    sid = lax.axis_index("s")
    bfs = [b0_v, b1_v, b2_v]
    msg = [m0_v, m1_v, m2_v]
    gsem = [g0, g1, g2]
    ssem = [s0, s1, s2]
    lo = cid * NHALF
    rbase = sid * R_PER_TILE
    crow0 = (cid * NS + sid) * CAP_CH
    zf = jnp.zeros((L,), jnp.float32)

    def zb(r, c2):
        for h in range(0, D, L):
            m0_v[r, pl.ds(h, L)] = zf
        return c2
    lax.fori_loop(0, RC, zb, 0)

    def cb(i, c2):
        pltpu.sync_copy(m0_v.at[pl.ds(0, RC)],
                        acc.at[pl.ds(rbase + i * RC, RC)])
        return c2
    lax.fori_loop(0, R_PER_TILE // RC, cb, 0)

    pltpu.sync_copy(counts.at[pl.ds((cid * NS + sid) * L, L)],
                    src_sv.at[pl.ds(0, L)])
    n_sup = lax.reduce_max(src_sv[pl.ds(0, L)], axes=(0,))
    plsc.subcore_barrier()

    def gissue(k, r):
        pltpu.async_copy(src_tab.at[src_sv.at[pl.ds(r * C, C)]],
                         bfs[k], gsem[k])

    def gwait(k):
        pltpu.make_async_copy(src_tab.at[src_sv.at[pl.ds(0, C)]],
                              bfs[k], gsem[k]).wait()

    def sissue(m, r):
        pltpu.async_copy(msg[m], acc.at[dst_sv.at[pl.ds(r * C, C)]],
                         ssem[m], add=True)

    def swait(m):
        pltpu.make_async_copy(msg[m], acc.at[dst_sv.at[pl.ds(0, C)]],
                              ssem[m]).wait()

    def scale(k, m, r):
        src = bfs[k]
        dst = msg[m]

        def gb(g, c2):
            vseg = val_sv[pl.ds(r * C + g * L, L)]
            for kk in range(L):
                v = vseg[kk]
                e = g * L + kk
                a0, a1 = plsc.unpack(src[e, pl.ds(0, 2 * L)], format=PK)
                b0, b1 = plsc.unpack(src[e, pl.ds(2 * L, 2 * L)], format=PK)
                dst[e, pl.ds(0, L)] = a0 * v
                dst[e, pl.ds(L, L)] = a1 * v
                dst[e, pl.ds(2 * L, L)] = b0 * v
                dst[e, pl.ds(3 * L, L)] = b1 * v
            return c2
        lax.fori_loop(0, C // L, gb, 0)

    def super_body(s, carry):
        e0 = (crow0 + s * SUP) * C
        pltpu.async_copy(csrc.at[pl.ds(e0, SUP * C)], src_sv, i_s)
        pltpu.async_copy(cdst.at[pl.ds(e0, SUP * C)], dst_sv, i_s)
        pltpu.async_copy(cval.at[pl.ds(e0, SUP * C)], val_sv, i_s)
        pltpu.make_async_copy(csrc.at[pl.ds(e0, SUP * C)], src_sv,
                              i_s).wait()
        pltpu.make_async_copy(cdst.at[pl.ds(e0, SUP * C)], dst_sv,
                              i_s).wait()
        pltpu.make_async_copy(cval.at[pl.ds(e0, SUP * C)], val_sv,
                              i_s).wait()
        for k in range(GLEAD):
            gissue(k, k)
        for t in range(SUP):
            k = t % NB
            m = t % NM
            ta = t + GLEAD
            if ta < SUP:
                gissue(ta % NB, ta)
            gwait(k)
            if t >= NM:
                swait(m)
            scale(k, m, t)
            sissue(m, t)
        for t in range(SUP - NM, SUP):
            swait(t % NM)
        return carry
    lax.fori_loop(0, n_sup, super_body, 0)
    plsc.subcore_barrier()

    if not mean_mode:
        def eb(i, c2):
            r0 = rbase + i * RC
            pltpu.sync_copy(acc.at[pl.ds(r0, RC)], m0_v.at[pl.ds(0, RC)])

            def pr(r, c3):
                _pack_row(m0_v, b0_v, r)
                return c3
            lax.fori_loop(0, RC, pr, 0)
            pltpu.sync_copy(b0_v.at[pl.ds(0, RC)],
                            out.at[pl.ds(lo + r0, RC)])
            return c2
        lax.fori_loop(0, R_PER_TILE // RC, eb, 0)
    else:
        quarter = jnp.float32(0.25)

        def mb(i, carry):
            r0 = rbase + i * RC
            g0r = lo + r0
            pltpu.sync_copy(acc.at[pl.ds(r0, RC)], m0_v.at[pl.ds(0, RC)])
            pltpu.sync_copy(buf1.at[pl.ds(g0r, RC)], b0_v.at[pl.ds(0, RC)])
            pltpu.sync_copy(src_tab.at[pl.ds(g0r, RC)], b1_v.at[pl.ds(0, RC)])

            def rb1(r, c2):
                for h in (0, 2 * L):
                    a0, a1 = plsc.unpack(b0_v[r, pl.ds(h, 2 * L)], format=PK)
                    c0, c1 = plsc.unpack(b1_v[r, pl.ds(h, 2 * L)], format=PK)
                    m0_v[r, pl.ds(h, L)] = m0_v[r, pl.ds(h, L)] + a0 + c0
                    m0_v[r, pl.ds(h + L, L)] = (m0_v[r, pl.ds(h + L, L)]
                                                + a1 + c1)
                return c2
            lax.fori_loop(0, RC, rb1, 0)
            pltpu.sync_copy(tab.at[pl.ds(g0r, RC)], m1_v.at[pl.ds(0, RC)])

            def rb2(r, c2):
                for h in range(0, D, L):
                    m0_v[r, pl.ds(h, L)] = (m0_v[r, pl.ds(h, L)]
                                            + m1_v[r, pl.ds(h, L)]) * quarter
                return c2
            lax.fori_loop(0, RC, rb2, 0)
            pltpu.sync_copy(m0_v.at[pl.ds(0, RC)], out.at[pl.ds(g0r, RC)])
            return carry
        lax.fori_loop(0, R_PER_TILE // RC, mb, 0)
  return _body


def _mesh():
    return plsc.VectorSubcoreMesh(core_axis_name="c", subcore_axis_name="s")


def _layer_scratch():
    f32 = jnp.float32
    i32 = jnp.int32
    bf16 = jnp.bfloat16
    return (
        [pltpu.VMEM_SHARED((NHALF, D), f32)]
        + [pltpu.VMEM((SUP * C,), i32),
           pltpu.VMEM((SUP * C,), i32),
           pltpu.VMEM((SUP * C,), f32)]
        + [pltpu.VMEM((C, D), bf16)] * NB
        + [pltpu.VMEM((C, D), f32)] * NM
        + [pltpu.SemaphoreType.DMA] * (NB + NM + 1)
    )


@jax.jit
def _run(tab, src2, dst2, val2):
    f32 = jnp.float32
    i32 = jnp.int32
    bf16 = jnp.bfloat16
    cp = pltpu.CompilerParams(use_tc_tiling_on_sc=False,
                              needs_layout_passes=False)
    nce = 2 * NS * CAP_CH * C
    p0 = pl.kernel(
        _p0_body,
        out_type=(jax.ShapeDtypeStruct((N_PAD, D), bf16),
                  jax.ShapeDtypeStruct((nce,), i32),
                  jax.ShapeDtypeStruct((nce,), i32),
                  jax.ShapeDtypeStruct((nce,), f32),
                  jax.ShapeDtypeStruct((2 * NS * L,), i32)),
        mesh=_mesh(),
        scratch_types=[pltpu.VMEM((SUP * C,), i32),
                       pltpu.VMEM((SUP * C,), i32),
                       pltpu.VMEM((SUP * C,), f32),
                       pltpu.VMEM((C, D), f32),
                       pltpu.VMEM((C, D), bf16),
                       pltpu.VMEM((STG,), i32),
                       pltpu.VMEM((STG,), i32),
                       pltpu.VMEM((STG,), f32),
                       pltpu.SemaphoreType.DMA],
        compiler_params=cp,
    )
    tb, csrc, cdst, cval, counts = p0(tab, src2, dst2, val2)

    layer = pl.kernel(
        _make_layer(False),
        out_type=jax.ShapeDtypeStruct((N_PAD, D), bf16),
        mesh=_mesh(),
        scratch_types=_layer_scratch(),
        compiler_params=cp,
    )
    buf1 = layer(tb, csrc, cdst, cval, counts)
    buf2 = layer(buf1, csrc, cdst, cval, counts)

    last = pl.kernel(
        _make_layer(True),
        out_type=jax.ShapeDtypeStruct((N_PAD, D), f32),
        mesh=_mesh(),
        scratch_types=_layer_scratch(),
        compiler_params=cp,
    )
    final = last(tab, buf1, buf2, csrc, cdst, cval, counts)
    return final


def kernel(adj_indices, adj_values, user_table, item_table):
    table = jnp.concatenate([user_table, item_table], axis=0)
    tab = jnp.pad(table, ((0, N_PAD - N), (0, 0)))

    E = adj_values.shape[0]
    e_block = NS * C * SUP
    E_pad = ((E + e_block - 1) // e_block) * e_block
    pad = E_pad - E
    dst2 = jnp.concatenate([adj_indices[0], jnp.zeros((pad,), jnp.int32)])
    src2 = jnp.concatenate([adj_indices[1], jnp.zeros((pad,), jnp.int32)])
    val2 = jnp.concatenate([adj_values, jnp.zeros((pad,), jnp.float32)])

    final = _run(tab, src2, dst2, val2)
    return (final[:N_USER], final[N_USER:N])


# speedup vs baseline: 1.3467x; 1.3467x over previous
import functools
import jax
import jax.numpy as jnp
from jax import lax
from jax.experimental import pallas as pl
from jax.experimental.pallas import tpu as pltpu
from jax.experimental.pallas import tpu_sc as plsc

N_USER = 25000
N_ITEM = 25000
N = N_USER + N_ITEM
D = 64
H = D // 2
NS = 16
L = 16
C = 128
SUP = 16
NBUF = 5
GLEAD = 3
N_PAD = 50176
R_PER_TILE = N_PAD // NS
RC = 112


def _make_sc_body(n_supers):
  def _sc_body(tab2, src2, dst2, val2,
               final, buf0, buf1,
               acc, src_sv, dst_sv, val_sv,
               r0_v, r1_v, r2_v, r3_v, r4_v,
               g0_s, g1_s, g2_s, g3_s, g4_s,
               s0_s, s1_s, s2_s, s3_s, s4_s, i_s):
    cid = lax.axis_index("c")
    sid = lax.axis_index("s")
    rows = [r0_v, r1_v, r2_v, r3_v, r4_v]
    gsem = [g0_s, g1_s, g2_s, g3_s, g4_s]
    ssem = [s0_s, s1_s, s2_s, s3_s, s4_s]
    zeros16 = jnp.zeros((L,), jnp.float32)
    row_off = cid * N_PAD
    rbase = sid * R_PER_TILE
    erow_base = sid * (n_supers * SUP)

    def fill_zero(buf):
        def zbody(r, carry):
            buf[r, pl.ds(0, L)] = zeros16
            buf[r, pl.ds(L, L)] = zeros16
            return carry
        lax.fori_loop(0, RC, zbody, 0)

    fill_zero(r0_v)

    def clear_acc(i, carry):
        pltpu.sync_copy(r0_v.at[pl.ds(0, RC)],
                        acc.at[pl.ds(rbase + i * RC, RC)])
        return carry
    lax.fori_loop(0, R_PER_TILE // RC, clear_acc, 0)
    plsc.subcore_barrier()

    def gissue(k, r, src_tab):
        pltpu.async_copy(src_tab.at[src_sv.at[r]], rows[k], gsem[k])

    def gwait(k, src_tab):
        pltpu.make_async_copy(src_tab.at[src_sv.at[0]], rows[k],
                              gsem[k]).wait()

    def sissue(k, r):
        pltpu.async_copy(rows[k], acc.at[dst_sv.at[r]], ssem[k], add=True)

    def swait(k):
        pltpu.make_async_copy(rows[k], acc.at[dst_sv.at[0]], ssem[k]).wait()

    def scale(k, r):
        buf = rows[k]

        def gb(g, carry):
            vseg = val_sv[r, pl.ds(g * L, L)]
            for kk in range(L):
                v = vseg[kk]
                e = g * L + kk
                buf[e, pl.ds(0, L)] = buf[e, pl.ds(0, L)] * v
                buf[e, pl.ds(L, L)] = buf[e, pl.ds(L, L)] * v
            return carry
        lax.fori_loop(0, C // L, gb, 0)

    def do_edges(src_tab):
        def super_body(s, carry):
            erow0 = erow_base + s * SUP
            pltpu.async_copy(src2.at[pl.ds(erow0, SUP)], src_sv, i_s)
            pltpu.async_copy(dst2.at[pl.ds(erow0, SUP)], dst_sv, i_s)
            pltpu.async_copy(val2.at[pl.ds(erow0, SUP)], val_sv, i_s)
            pltpu.make_async_copy(src2.at[pl.ds(erow0, SUP)], src_sv,
                                  i_s).wait()
            pltpu.make_async_copy(dst2.at[pl.ds(erow0, SUP)], dst_sv,
                                  i_s).wait()
            pltpu.make_async_copy(val2.at[pl.ds(erow0, SUP)], val_sv,
                                  i_s).wait()

            def adj_body(r, c2):
                for j in range(C // L):
                    src_sv[r, pl.ds(j * L, L)] = (
                        src_sv[r, pl.ds(j * L, L)] + row_off)
                return c2
            lax.fori_loop(0, SUP, adj_body, 0)

            for k in range(GLEAD):
                gissue(k, k, src_tab)

            for t in range(SUP):
                k = t % NBUF
                ta = t + GLEAD
                if ta < SUP:
                    kb = ta % NBUF
                    if ta - NBUF >= 0:
                        swait(kb)
                    gissue(kb, ta, src_tab)
                gwait(k, src_tab)
                scale(k, t)
                sissue(k, t)
            for t in range(SUP - NBUF, SUP):
                swait(t % NBUF)
            return carry
        lax.fori_loop(0, n_supers, super_body, 0)
        plsc.subcore_barrier()

    def export_layer(dst_buf):
        fill_zero(r1_v)

        def eb(i, carry):
            r0 = rbase + i * RC
            pltpu.sync_copy(acc.at[pl.ds(r0, RC)], r0_v.at[pl.ds(0, RC)])
            pltpu.sync_copy(r0_v.at[pl.ds(0, RC)],
                            dst_buf.at[pl.ds(row_off + r0, RC)])
            pltpu.sync_copy(r1_v.at[pl.ds(0, RC)], acc.at[pl.ds(r0, RC)])
            return carry
        lax.fori_loop(0, R_PER_TILE // RC, eb, 0)
        plsc.subcore_barrier()

    do_edges(tab2)
    export_layer(buf0)
    do_edges(buf0)
    export_layer(buf1)
    do_edges(buf1)

    quarter = jnp.float32(0.25)

    def mean_body(i, carry):
        r0 = rbase + i * RC
        g0 = row_off + r0
        pltpu.sync_copy(acc.at[pl.ds(r0, RC)], r0_v.at[pl.ds(0, RC)])
        pltpu.sync_copy(tab2.at[pl.ds(g0, RC)], r1_v.at[pl.ds(0, RC)])
        pltpu.sync_copy(buf0.at[pl.ds(g0, RC)], r2_v.at[pl.ds(0, RC)])
        pltpu.sync_copy(buf1.at[pl.ds(g0, RC)], r3_v.at[pl.ds(0, RC)])

        def rbody(r, rcarry):
            for h in (0, L):
                s = (r0_v[r, pl.ds(h, L)] + r1_v[r, pl.ds(h, L)]
                     + r2_v[r, pl.ds(h, L)] + r3_v[r, pl.ds(h, L)])
                r0_v[r, pl.ds(h, L)] = s * quarter
            return rcarry
        lax.fori_loop(0, RC, rbody, 0)
        pltpu.sync_copy(r0_v.at[pl.ds(0, RC)], final.at[pl.ds(g0, RC)])
        return carry
    lax.fori_loop(0, R_PER_TILE // RC, mean_body, 0)

  return _sc_body


@functools.partial(jax.jit, static_argnames=("n_supers",))
def _run(tab2, src2, dst2, val2, n_supers):
    mesh = plsc.VectorSubcoreMesh(core_axis_name="c", subcore_axis_name="s")
    f32 = jnp.float32
    i32 = jnp.int32
    out_type = (
        jax.ShapeDtypeStruct((2 * N_PAD, H), f32),
        jax.ShapeDtypeStruct((2 * N_PAD, H), f32),
        jax.ShapeDtypeStruct((2 * N_PAD, H), f32),
    )
    scratch = (
        [pltpu.VMEM_SHARED((N_PAD, H), f32)]
        + [pltpu.VMEM((SUP, C), i32),
           pltpu.VMEM((SUP, C), i32),
           pltpu.VMEM((SUP, C), f32)]
        + [pltpu.VMEM((C, H), f32)] * NBUF
        + [pltpu.SemaphoreType.DMA] * (2 * NBUF + 1)
    )
    run = pl.kernel(
        _make_sc_body(n_supers),
        out_type=out_type,
        mesh=mesh,
        scratch_types=scratch,
        compiler_params=pltpu.CompilerParams(use_tc_tiling_on_sc=False),
    )
    final, _, _ = run(tab2, src2, dst2, val2)
    return final


def kernel(adj_indices, adj_values, user_table, item_table):
    table = jnp.concatenate([user_table, item_table], axis=0)
    table = jnp.pad(table, ((0, N_PAD - N), (0, 0)))
    tab2 = jnp.concatenate([table[:, :H], table[:, H:]], axis=0)

    E = adj_values.shape[0]
    e_block = NS * C * SUP
    E_pad = ((E + e_block - 1) // e_block) * e_block
    pad = E_pad - E
    dst2 = jnp.concatenate(
        [adj_indices[0], jnp.zeros((pad,), jnp.int32)]).reshape(-1, C)
    src2 = jnp.concatenate(
        [adj_indices[1], jnp.zeros((pad,), jnp.int32)]).reshape(-1, C)
    val2 = jnp.concatenate(
        [adj_values, jnp.zeros((pad,), jnp.float32)]).reshape(-1, C)

    final = _run(tab2, src2, dst2, val2, E_pad // e_block)
    all_embed = jnp.concatenate([final[:N], final[N_PAD:N_PAD + N]], axis=1)
    return (all_embed[:N_USER], all_embed[N_USER:])
